# baseline (speedup 1.0000x reference)
import dataclasses
import functools

import jax
import jax.numpy as jnp
from jax import lax
from jax.experimental import pallas as pl
from jax.experimental.pallas import tpu as pltpu
from jax.experimental.pallas import tpu_sc as plsc

MARGIN = 0.2
BATCH = 16384
DIM = 128
N_CLASSES = 1000

NC = 2
NS = 16
NW = NC * NS
TRIP_PER_W = BATCH // NW
NGROUP = TRIP_PER_W // 16
CHUNK = 256
NCHUNK = 3 * TRIP_PER_W // CHUNK

R = 2048
NB = BATCH // R


def _sc_gather(batch, idx_all, labels, beta):
    mesh = plsc.VectorSubcoreMesh(core_axis_name="c", subcore_axis_name="s")
    cp = pltpu.CompilerParams()
    if "needs_layout_passes" in pltpu.CompilerParams.__dataclass_fields__:
        cp = dataclasses.replace(cp, needs_layout_passes=False)

    @functools.partial(
        pl.kernel,
        compiler_params=cp,
        out_type=(
            jax.ShapeDtypeStruct((3 * BATCH, DIM), jnp.float32),
            jax.ShapeDtypeStruct((BATCH,), jnp.float32),
        ),
        mesh=mesh,
        scratch_types=[
            pltpu.VMEM((3 * TRIP_PER_W,), jnp.int32),
            pltpu.VMEM((CHUNK, DIM), jnp.float32),
            pltpu.VMEM((CHUNK, DIM), jnp.float32),
            pltpu.VMEM((BATCH,), jnp.int32),
            pltpu.VMEM((N_CLASSES,), jnp.float32),
            pltpu.VMEM((TRIP_PER_W,), jnp.float32),
            pltpu.SemaphoreType.DMA,
            pltpu.SemaphoreType.DMA,
        ],
    )
    def k(batch_hbm, idx_hbm, labels_hbm, beta_hbm, rows_out, beta_t_out,
          idx_v, rows0_v, rows1_v, labels_v, beta_v, bt_v, sem_g, sem_w):
        wid = lax.axis_index("s") * NC + lax.axis_index("c")
        tbase = wid * TRIP_PER_W
        bufs = (rows0_v, rows1_v)

        for c in range(3):
            pltpu.sync_copy(idx_hbm.at[pl.ds(c * BATCH + tbase, TRIP_PER_W)],
                            idx_v.at[pl.ds(c * TRIP_PER_W, TRIP_PER_W)])

        def idx_slice(k):
            return idx_v.at[pl.ds(k * CHUNK, CHUNK)]

        def out_slice(k):
            c, h = divmod(k, 3 * TRIP_PER_W // CHUNK // 3)
            return rows_out.at[pl.ds(c * BATCH + tbase + h * CHUNK, CHUNK)]

        gathers = [None] * NCHUNK
        writes = [None] * NCHUNK
        gathers[0] = pltpu.async_copy(batch_hbm.at[idx_slice(0)], bufs[0], sem_g)

        pltpu.sync_copy(labels_hbm, labels_v)
        pltpu.sync_copy(beta_hbm, beta_v)

        @pl.loop(0, NGROUP)
        def _(g):
            t0 = idx_v[pl.ds(g * 16, 16)]
            la = plsc.load_gather(labels_v, [t0])
            bt_v[pl.ds(g * 16, 16)] = plsc.load_gather(beta_v, [la])

        pltpu.sync_copy(bt_v, beta_t_out.at[pl.ds(tbase, TRIP_PER_W)])

        for k in range(NCHUNK):
            buf = bufs[k % 2]
            gathers[k].wait()
            writes[k] = pltpu.async_copy(buf, out_slice(k), sem_w)
            if k + 1 < NCHUNK:
                if k >= 1:
                    writes[k - 1].wait()
                gathers[k + 1] = pltpu.async_copy(
                    batch_hbm.at[idx_slice(k + 1)], bufs[(k + 1) % 2], sem_g)
        writes[NCHUNK - 2].wait()
        writes[NCHUNK - 1].wait()

    return k(batch, idx_all, labels, beta)


def _tc_reduce_body(a_ref, p_ref, n_ref, bt_ref, out_ref):
    a = a_ref[...]
    p = p_ref[...]
    n = n_ref[...]
    bt = bt_ref[0, 0]
    dap = a - p
    dan = a - n
    sq = jnp.concatenate([dap * dap, dan * dan], axis=0)
    d2 = jnp.sum(sq.T, axis=0)
    d = jnp.sqrt(d2 + 1e-8)
    pos = jnp.maximum(d[:R] - bt + MARGIN, 0.0)
    neg = jnp.maximum(bt - d[R:] + MARGIN, 0.0)
    out_ref[0, 0, 0] = jnp.sum(pos + neg)
    out_ref[0, 0, 1] = jnp.sum((pos > 0.0).astype(jnp.float32)
                               + (neg > 0.0).astype(jnp.float32))


def _tc_final_body(part_ref, out_ref):
    tot = 0.0
    cnt = 0.0
    for j in range(NB):
        tot += part_ref[j, 0, 0]
        cnt += part_ref[j, 0, 1]
    out_ref[0, 0] = jnp.where(cnt == 0.0, tot, tot / jnp.maximum(cnt, 1.0))


def _tc_reduce(rows, beta_t):
    bt3 = beta_t.reshape(NB, 1, R)
    parts = pl.pallas_call(
        _tc_reduce_body,
        grid=(NB,),
        in_specs=[
            pl.BlockSpec((R, DIM), lambda i: (i, 0)),
            pl.BlockSpec((R, DIM), lambda i: (i + NB, 0)),
            pl.BlockSpec((R, DIM), lambda i: (i + 2 * NB, 0)),
            pl.BlockSpec((1, 1, R), lambda i: (i, 0, 0)),
        ],
        out_specs=pl.BlockSpec((1, 1, 2), lambda i: (i, 0, 0),
                               memory_space=pltpu.SMEM),
        out_shape=jax.ShapeDtypeStruct((NB, 1, 2), jnp.float32),
        compiler_params=pltpu.CompilerParams(
            dimension_semantics=("parallel",)),
    )(rows, rows, rows, bt3)
    return pl.pallas_call(
        _tc_final_body,
        in_specs=[pl.BlockSpec(memory_space=pltpu.SMEM)],
        out_specs=pl.BlockSpec(memory_space=pltpu.SMEM),
        out_shape=jax.ShapeDtypeStruct((1, 1), jnp.float32),
    )(parts)


def kernel(batch, beta, labels, triplets):
    idx_all = jnp.transpose(triplets).reshape(3 * BATCH)
    rows, beta_t = _sc_gather(batch, idx_all, labels, beta)
    loss = _tc_reduce(rows, beta_t)
    return loss[0, 0]


# speedup vs baseline: 1.0255x; 1.0255x over previous
import dataclasses
import functools

import jax
import jax.numpy as jnp
from jax import lax
from jax.experimental import pallas as pl
from jax.experimental.pallas import tpu as pltpu
from jax.experimental.pallas import tpu_sc as plsc

MARGIN = 0.2
BATCH = 16384
DIM = 128
N_CLASSES = 1000

NC = 2
NS = 16
NW = NC * NS
TRIP_PER_W = BATCH // NW
NGROUP = TRIP_PER_W // 16
CHUNK = 256
NCHUNK = 3 * TRIP_PER_W // CHUNK

R = 2048
NB = BATCH // R


def _sc_gather(batch, idx_all, labels, beta):
    mesh = plsc.VectorSubcoreMesh(core_axis_name="c", subcore_axis_name="s")
    cp = pltpu.CompilerParams()
    if "needs_layout_passes" in pltpu.CompilerParams.__dataclass_fields__:
        cp = dataclasses.replace(cp, needs_layout_passes=False)

    @functools.partial(
        pl.kernel,
        compiler_params=cp,
        out_type=(
            jax.ShapeDtypeStruct((3 * BATCH, DIM), jnp.float32),
            jax.ShapeDtypeStruct((BATCH,), jnp.float32),
        ),
        mesh=mesh,
        scratch_types=[
            pltpu.VMEM((3 * TRIP_PER_W,), jnp.int32),
            pltpu.VMEM((CHUNK, DIM), jnp.float32),
            pltpu.VMEM((CHUNK, DIM), jnp.float32),
            pltpu.VMEM((BATCH,), jnp.int32),
            pltpu.VMEM((N_CLASSES,), jnp.float32),
            pltpu.VMEM((TRIP_PER_W,), jnp.float32),
            pltpu.SemaphoreType.DMA,
            pltpu.SemaphoreType.DMA,
        ],
    )
    def k(batch_hbm, idx_hbm, labels_hbm, beta_hbm, rows_out, beta_t_out,
          idx_v, rows0_v, rows1_v, labels_v, beta_v, bt_v, sem_g, sem_w):
        wid = lax.axis_index("s") * NC + lax.axis_index("c")
        tbase = wid * TRIP_PER_W
        bufs = (rows0_v, rows1_v)

        for c in range(3):
            pltpu.sync_copy(idx_hbm.at[pl.ds(c * BATCH + tbase, TRIP_PER_W)],
                            idx_v.at[pl.ds(c * TRIP_PER_W, TRIP_PER_W)])

        def idx_slice(k):
            return idx_v.at[pl.ds(k * CHUNK, CHUNK)]

        def out_slice(k):
            c, h = divmod(k, 3 * TRIP_PER_W // CHUNK // 3)
            sl = tbase // R
            off = tbase % R + h * CHUNK
            return rows_out.at[pl.ds(sl * (3 * R) + c * R + off, CHUNK)]

        gathers = [None] * NCHUNK
        writes = [None] * NCHUNK
        gathers[0] = pltpu.async_copy(batch_hbm.at[idx_slice(0)], bufs[0], sem_g)

        pltpu.sync_copy(labels_hbm, labels_v)
        pltpu.sync_copy(beta_hbm, beta_v)

        @pl.loop(0, NGROUP)
        def _(g):
            t0 = idx_v[pl.ds(g * 16, 16)]
            la = plsc.load_gather(labels_v, [t0])
            bt_v[pl.ds(g * 16, 16)] = plsc.load_gather(beta_v, [la])

        pltpu.sync_copy(bt_v, beta_t_out.at[pl.ds(tbase, TRIP_PER_W)])

        for k in range(NCHUNK):
            buf = bufs[k % 2]
            gathers[k].wait()
            writes[k] = pltpu.async_copy(buf, out_slice(k), sem_w)
            if k + 1 < NCHUNK:
                if k >= 1:
                    writes[k - 1].wait()
                gathers[k + 1] = pltpu.async_copy(
                    batch_hbm.at[idx_slice(k + 1)], bufs[(k + 1) % 2], sem_g)
        writes[NCHUNK - 2].wait()
        writes[NCHUNK - 1].wait()

    return k(batch, idx_all, labels, beta)


def _tc_reduce_body(x_ref, bt_ref, out_ref, acc_ref):
    i = pl.program_id(0)

    @pl.when(i == 0)
    def _():
        acc_ref[0] = 0.0
        acc_ref[1] = 0.0

    x = x_ref[...]
    a = x[:R]
    p = x[R:2 * R]
    n = x[2 * R:]
    bt = bt_ref[0, 0]
    dap = a - p
    dan = a - n
    sq = jnp.concatenate([dap * dap, dan * dan], axis=0)
    d2 = jnp.sum(sq.T, axis=0)
    d = jnp.sqrt(d2 + 1e-8)
    pos = jnp.maximum(d[:R] - bt + MARGIN, 0.0)
    neg = jnp.maximum(bt - d[R:] + MARGIN, 0.0)
    acc_ref[0] += jnp.sum(pos + neg)
    acc_ref[1] += jnp.sum((pos > 0.0).astype(jnp.float32)
                          + (neg > 0.0).astype(jnp.float32))

    @pl.when(i == NB - 1)
    def _():
        tot = acc_ref[0]
        cnt = acc_ref[1]
        out_ref[0, 0] = jnp.where(cnt == 0.0, tot, tot / jnp.maximum(cnt, 1.0))


def _tc_reduce(rows, beta_t):
    bt3 = beta_t.reshape(NB, 1, R)
    return pl.pallas_call(
        _tc_reduce_body,
        grid=(NB,),
        in_specs=[
            pl.BlockSpec((3 * R, DIM), lambda i: (i, 0)),
            pl.BlockSpec((1, 1, R), lambda i: (i, 0, 0)),
        ],
        out_specs=pl.BlockSpec(memory_space=pltpu.SMEM),
        out_shape=jax.ShapeDtypeStruct((1, 1), jnp.float32),
        scratch_shapes=[pltpu.SMEM((2,), jnp.float32)],
    )(rows, bt3)


def kernel(batch, beta, labels, triplets):
    idx_all = jnp.transpose(triplets).reshape(3 * BATCH)
    rows, beta_t = _sc_gather(batch, idx_all, labels, beta)
    loss = _tc_reduce(rows, beta_t)
    return loss[0, 0]


# speedup vs baseline: 1.0610x; 1.0347x over previous
import dataclasses
import functools

import jax
import jax.numpy as jnp
from jax import lax
from jax.experimental import pallas as pl
from jax.experimental.pallas import tpu as pltpu
from jax.experimental.pallas import tpu_sc as plsc

MARGIN = 0.2
BATCH = 16384
DIM = 128
N_CLASSES = 1000

NC = 2
NS = 16
NW = NC * NS
TRIP_PER_W = BATCH // NW
NGROUP = TRIP_PER_W // 16
CHUNK = 256
NCHUNK = 3 * TRIP_PER_W // CHUNK

R = 4096
NB = BATCH // R


def _sc_gather(batch, idx_all, labels, beta):
    mesh = plsc.VectorSubcoreMesh(core_axis_name="c", subcore_axis_name="s")
    cp = pltpu.CompilerParams()
    if "needs_layout_passes" in pltpu.CompilerParams.__dataclass_fields__:
        cp = dataclasses.replace(cp, needs_layout_passes=False)

    @functools.partial(
        pl.kernel,
        compiler_params=cp,
        out_type=(
            jax.ShapeDtypeStruct((3 * BATCH, DIM), jnp.float32),
            jax.ShapeDtypeStruct((BATCH,), jnp.float32),
        ),
        mesh=mesh,
        scratch_types=[
            pltpu.VMEM((3 * TRIP_PER_W,), jnp.int32),
            pltpu.VMEM((CHUNK, DIM), jnp.float32),
            pltpu.VMEM((CHUNK, DIM), jnp.float32),
            pltpu.VMEM((BATCH,), jnp.int32),
            pltpu.VMEM((N_CLASSES,), jnp.float32),
            pltpu.VMEM((TRIP_PER_W,), jnp.float32),
            pltpu.SemaphoreType.DMA,
            pltpu.SemaphoreType.DMA,
        ],
    )
    def k(batch_hbm, idx_hbm, labels_hbm, beta_hbm, rows_out, beta_t_out,
          idx_v, rows0_v, rows1_v, labels_v, beta_v, bt_v, sem_g, sem_w):
        wid = lax.axis_index("s") * NC + lax.axis_index("c")
        tbase = wid * TRIP_PER_W
        bufs = (rows0_v, rows1_v)

        for c in range(3):
            pltpu.sync_copy(idx_hbm.at[pl.ds(c * BATCH + tbase, TRIP_PER_W)],
                            idx_v.at[pl.ds(c * TRIP_PER_W, TRIP_PER_W)])

        def idx_slice(k):
            return idx_v.at[pl.ds(k * CHUNK, CHUNK)]

        def out_slice(k):
            c, h = divmod(k, 3 * TRIP_PER_W // CHUNK // 3)
            sl = tbase // R
            off = tbase % R + h * CHUNK
            return rows_out.at[pl.ds(sl * (3 * R) + c * R + off, CHUNK)]

        gathers = [None] * NCHUNK
        writes = [None] * NCHUNK
        gathers[0] = pltpu.async_copy(batch_hbm.at[idx_slice(0)], bufs[0], sem_g)

        pltpu.sync_copy(labels_hbm, labels_v)
        pltpu.sync_copy(beta_hbm, beta_v)

        @pl.loop(0, NGROUP)
        def _(g):
            t0 = idx_v[pl.ds(g * 16, 16)]
            la = plsc.load_gather(labels_v, [t0])
            bt_v[pl.ds(g * 16, 16)] = plsc.load_gather(beta_v, [la])

        pltpu.sync_copy(bt_v, beta_t_out.at[pl.ds(tbase, TRIP_PER_W)])

        for k in range(NCHUNK):
            buf = bufs[k % 2]
            gathers[k].wait()
            writes[k] = pltpu.async_copy(buf, out_slice(k), sem_w)
            if k + 1 < NCHUNK:
                if k >= 1:
                    writes[k - 1].wait()
                gathers[k + 1] = pltpu.async_copy(
                    batch_hbm.at[idx_slice(k + 1)], bufs[(k + 1) % 2], sem_g)
        writes[NCHUNK - 2].wait()
        writes[NCHUNK - 1].wait()

    return k(batch, idx_all, labels, beta)


def _tc_reduce_body(x_ref, bt_ref, out_ref, acc_ref):
    i = pl.program_id(0)

    @pl.when(i == 0)
    def _():
        acc_ref[0] = 0.0
        acc_ref[1] = 0.0

    x = x_ref[...]
    a = x[:R]
    p = x[R:2 * R]
    n = x[2 * R:]
    bt = bt_ref[0, 0]
    dap = a - p
    dan = a - n
    sq = jnp.concatenate([dap * dap, dan * dan], axis=0)
    d2 = jnp.sum(sq.T, axis=0)
    d = jnp.sqrt(d2 + 1e-8)
    pos = jnp.maximum(d[:R] - bt + MARGIN, 0.0)
    neg = jnp.maximum(bt - d[R:] + MARGIN, 0.0)
    acc_ref[0] += jnp.sum(pos + neg)
    acc_ref[1] += jnp.sum((pos > 0.0).astype(jnp.float32)
                          + (neg > 0.0).astype(jnp.float32))

    @pl.when(i == NB - 1)
    def _():
        tot = acc_ref[0]
        cnt = acc_ref[1]
        out_ref[0, 0] = jnp.where(cnt == 0.0, tot, tot / jnp.maximum(cnt, 1.0))


def _tc_reduce(rows, beta_t):
    bt3 = beta_t.reshape(NB, 1, R)
    return pl.pallas_call(
        _tc_reduce_body,
        grid=(NB,),
        in_specs=[
            pl.BlockSpec((3 * R, DIM), lambda i: (i, 0)),
            pl.BlockSpec((1, 1, R), lambda i: (i, 0, 0)),
        ],
        out_specs=pl.BlockSpec(memory_space=pltpu.SMEM),
        out_shape=jax.ShapeDtypeStruct((1, 1), jnp.float32),
        scratch_shapes=[pltpu.SMEM((2,), jnp.float32)],
    )(rows, bt3)


def kernel(batch, beta, labels, triplets):
    idx_all = jnp.transpose(triplets).reshape(3 * BATCH)
    rows, beta_t = _sc_gather(batch, idx_all, labels, beta)
    loss = _tc_reduce(rows, beta_t)
    return loss[0, 0]
